# Initial kernel scaffold; baseline (speedup 1.0000x reference)
import jax
import jax.numpy as jnp
from jax.experimental import pallas as pl


def kernel(x, table):
    raise NotImplementedError("write your pallas kernel here")


# speedup vs baseline: 32.1075x; 32.1075x over previous
import functools

import numpy as np
import jax
import jax.numpy as jnp
from jax import lax
from jax.experimental import pallas as pl
from jax.experimental.pallas import tpu as pltpu
from jax.experimental.pallas import tpu_sc as plsc

_N_LEVELS = 16
_LOG2_T = 19
_T = 1 << _LOG2_T
_MASK = _T - 1
_BASE_RES = 16
_FINEST_RES = 512
_B = (_FINEST_RES / _BASE_RES) ** (1.0 / (_N_LEVELS - 1))
_RES = [int(np.floor(_BASE_RES * (_B ** l))) for l in range(_N_LEVELS)]
_P1 = np.int32(np.uint32(2654435761).astype(np.int32))
_P2 = np.int32(805459861)
_L = 16


def _build(n_points, chunk, num_cores=2, num_subcores=16, interpret=False):
    nw = num_cores * num_subcores
    pw = n_points // nw
    nch = pw // chunk
    ng = chunk // _L
    nrow = 8 * chunk
    niw = nrow // 128
    mesh = plsc.VectorSubcoreMesh(
        core_axis_name="c", subcore_axis_name="s",
        num_cores=num_cores, num_subcores=num_subcores)

    @functools.partial(
        pl.kernel,
        out_type=jax.ShapeDtypeStruct((n_points, 2 * _N_LEVELS), jnp.float32),
        mesh=mesh,
        interpret=interpret,
        compiler_params=pltpu.CompilerParams(
            needs_layout_passes=False, use_tc_tiling_on_sc=False),
        scratch_types=[
            pltpu.VMEM((chunk, 3), jnp.float32),
            pltpu.VMEM((nrow,), jnp.int32),
            pltpu.VMEM((nrow,), jnp.int32),
            pltpu.VMEM((nrow,), jnp.float32),
            pltpu.VMEM((nrow,), jnp.float32),
            pltpu.VMEM((nrow, 2), jnp.float32),
            pltpu.VMEM((nrow, 2), jnp.float32),
            pltpu.VMEM((chunk, 2 * _N_LEVELS), jnp.float32),
            pltpu.SemaphoreType.DMA,
            pltpu.SemaphoreType.DMA,
        ],
    )
    def grid_kernel(x_hbm, tbl_hbm, out_hbm, x_v, idx_v0, idx_v1, w_v0, w_v1,
                    rows_v0, rows_v1, out_v, sem0, sem1):
        idx_b = (idx_v0, idx_v1)
        w_b = (w_v0, w_v1)
        rows_b = (rows_v0, rows_v1)
        sems = (sem0, sem1)
        wid = lax.axis_index("s") * num_cores + lax.axis_index("c")
        iota = lax.iota(jnp.int32, _L)
        zeros = jnp.zeros((_L,), jnp.int32)
        ones = jnp.full((_L,), 1, jnp.int32)

        def idx_pass(l, par):
            res = float(_RES[l])
            lvl_base = l * _T

            @pl.loop(0, ng)
            def _g(g):
                rowi = g * _L + iota
                wofs = g * (8 * _L) + iota
                xd = [plsc.load_gather(x_v, [rowi, jnp.full((_L,), d, jnp.int32)])
                      for d in range(3)]
                pos = [(x * 0.5 + 0.5) * res for x in xd]
                ip = [p.astype(jnp.int32) for p in pos]
                fr = [p - i.astype(jnp.float32) for p, i in zip(pos, ip)]
                ix, iy, iz = ip
                by = (iy * _P1, (iy + 1) * _P1)
                cz = (iz * _P2, (iz + 1) * _P2)
                exy = [[ix ^ by[0], (ix + 1) ^ by[0]],
                       [ix ^ by[1], (ix + 1) ^ by[1]]]
                fx, fy, fz = fr
                gx, gy, gz = 1.0 - fx, 1.0 - fy, 1.0 - fz
                wxy = [[gx * gy, fx * gy], [gx * fy, fx * fy]]
                for j in range(8):
                    jx, jy, jz = j & 1, (j >> 1) & 1, (j >> 2) & 1
                    h = exy[jy][jx] ^ cz[jz]
                    idx = (h & _MASK) + lvl_base
                    plsc.store_scatter(idx_b[par], [wofs + j * _L], idx)
                    w = wxy[jy][jx] * (fz if jz else gz)
                    plsc.store_scatter(w_b[par], [wofs + j * _L], w)

        def fma_pass(l, par):
            @pl.loop(0, ng)
            def _g(g):
                wofs = g * (8 * _L) + iota
                acc0 = jnp.zeros((_L,), jnp.float32)
                acc1 = jnp.zeros((_L,), jnp.float32)
                for j in range(8):
                    r = wofs + j * _L
                    f0 = plsc.load_gather(rows_b[par], [r, zeros])
                    f1 = plsc.load_gather(rows_b[par], [r, ones])
                    w = plsc.load_gather(w_b[par], [wofs + j * _L])
                    acc0 = acc0 + f0 * w
                    acc1 = acc1 + f1 * w
                rowi = g * _L + iota
                plsc.store_scatter(
                    out_v, [rowi, jnp.full((_L,), 2 * l, jnp.int32)], acc0)
                plsc.store_scatter(
                    out_v, [rowi, jnp.full((_L,), 2 * l + 1, jnp.int32)], acc1)

        def fire(par):
            return pltpu.async_copy(
                tbl_hbm.at[idx_b[par]], rows_b[par], sems[par])

        @pl.loop(0, nch)
        def _chunk(c):
            pbase = wid * pw + c * chunk
            pltpu.sync_copy(x_hbm.at[pl.ds(pbase, chunk)], x_v)
            idx_pass(0, 0)
            h = fire(0)
            for l in range(1, _N_LEVELS):
                par = l % 2
                idx_pass(l, par)
                h_new = fire(par)
                h.wait()
                fma_pass(l - 1, 1 - par)
                h = h_new
            h.wait()
            fma_pass(_N_LEVELS - 1, (_N_LEVELS - 1) % 2)
            pltpu.sync_copy(out_v, out_hbm.at[pl.ds(pbase, chunk)])

    return grid_kernel


_kernel_impl = None


def kernel(x, table):
    global _kernel_impl
    n = x.shape[0]
    if _kernel_impl is None:
        _kernel_impl = _build(n, 512)
    tbl = table.reshape(_N_LEVELS * _T, 2)
    return _kernel_impl(x, tbl)
